# Initial kernel scaffold; baseline (speedup 1.0000x reference)
import jax
import jax.numpy as jnp
from jax.experimental import pallas as pl


def kernel(x_ind, x_org, x_ext, edge_indexes, edge_attrs, params):
    raise NotImplementedError("write your pallas kernel here")


# speedup vs baseline: 4.7780x; 4.7780x over previous
import functools

import jax
import jax.numpy as jnp
from jax import lax
from jax.experimental import pallas as pl
from jax.experimental.pallas import tpu as pltpu
from jax.experimental.pallas import tpu_sc as plsc

N_NODE = 10000
E = 160000
D = 256
DE = 16
NREL = 7
NC, NS, L = 2, 16, 16
NW = NC * NS
EPAD = 163840
PER_W = EPAD // NW
CE = 1024
NCHUNK = PER_W // CE
GPC = CE // L
AGG = 71680
DEAD = 70000

REL_SRC_ID = (0, 1, 2, 0, 1, 2, 1)
REL_SLOT = (0, 0, 0, 1, 1, 1, 2)
WCOLS = 48


def _mm_body(x_ref, w_ref, o_ref):
    o_ref[...] = lax.dot_general(
        x_ref[...], w_ref[...], (((1,), (0,)), ((), ())),
        preferred_element_type=jnp.float32,
        precision=lax.Precision.HIGHEST)


def _mm(x, w):
    br = 2000
    return pl.pallas_call(
        _mm_body,
        grid=(N_NODE // br,),
        in_specs=[pl.BlockSpec((br, D), lambda r: (r, 0)),
                  pl.BlockSpec((D, WCOLS), lambda r: (0, 0))],
        out_specs=pl.BlockSpec((br, WCOLS), lambda r: (r, 0)),
        out_shape=jax.ShapeDtypeStruct((N_NODE, WCOLS), jnp.float32),
    )(x, w)


def _edge_body(ytab, srcs, dsts, eids, ea0, ea1, ea2, ea3, ea4, ea5, ea6p,
               out, src_v, dst_v, eid_v, ea_v, ea6_v, rows_v, agg_v,
               sem1, sem2):
    eas = (ea0, ea1, ea2, ea3, ea4, ea5, ea6p)
    c = lax.axis_index("c")
    s = lax.axis_index("s")
    wid = s * NC + c
    iota = lax.iota(jnp.int32, L)

    def zero_body(g, carry):
        agg_v[pl.ds(g * L, L)] = jnp.zeros((L,), jnp.float32)
        return carry
    lax.fori_loop(0, AGG // L, zero_body, 0)

    for k in range(NREL):
        ea_ref = eas[k]

        def chunk_body(ci, carry, k=k, ea_ref=ea_ref):
            base = wid * PER_W + ci * CE
            pltpu.sync_copy(srcs.at[pl.ds(k * EPAD + base, CE)], src_v)
            pltpu.sync_copy(dsts.at[pl.ds(k * EPAD + base, CE)], dst_v)
            if k < 6:
                pltpu.sync_copy(eids.at[pl.ds(base, CE)], eid_v)
                cp_y = pltpu.async_copy(ytab.at[src_v], rows_v, sem1)
                cp_e = pltpu.async_copy(ea_ref.at[eid_v], ea_v, sem2)
                cp_y.wait()
                cp_e.wait()
            else:
                pltpu.sync_copy(ea_ref.at[pl.ds(base, CE)], ea6_v)
                pltpu.async_copy(ytab.at[src_v], rows_v, sem1).wait()

            def group_body(g, carry2, k=k):
                row0 = g * L
                ridx = row0 + iota
                if k < 6:
                    acc = jnp.zeros((L,), jnp.float32)
                    for j in range(DE):
                        cidx = jnp.full((L,), j, jnp.int32)
                        yv = plsc.load_gather(rows_v, [ridx, cidx])
                        ev = plsc.load_gather(ea_v, [ridx, cidx])
                        acc = acc + yv * ev
                else:
                    cidx = jnp.zeros((L,), jnp.int32)
                    yv = plsc.load_gather(rows_v, [ridx, cidx])
                    acc = yv * ea6_v[pl.ds(row0, L)]
                dv = dst_v[pl.ds(row0, L)]
                plsc.addupdate_scatter(agg_v, [dv], acc)
                return carry2
            lax.fori_loop(0, GPC, group_body, 0)
            return carry
        lax.fori_loop(0, NCHUNK, chunk_body, 0)

    pltpu.sync_copy(agg_v, out.at[wid])


def _edge_kernel(ytab, srcs, dsts, eids, ea_list, ea6p):
    mesh = plsc.VectorSubcoreMesh(
        core_axis_name="c", subcore_axis_name="s",
        num_cores=NC, num_subcores=NS)
    f = pl.kernel(
        _edge_body,
        out_type=jax.ShapeDtypeStruct((NW, AGG), jnp.float32),
        mesh=mesh,
        scratch_types=[
            pltpu.VMEM((CE,), jnp.int32),
            pltpu.VMEM((CE,), jnp.int32),
            pltpu.VMEM((CE,), jnp.int32),
            pltpu.VMEM((CE, DE), jnp.float32),
            pltpu.VMEM((CE,), jnp.float32),
            pltpu.VMEM((CE, DE), jnp.float32),
            pltpu.VMEM((AGG,), jnp.float32),
            pltpu.SemaphoreType.DMA,
            pltpu.SemaphoreType.DMA,
        ],
        compiler_params=pltpu.CompilerParams(
            needs_layout_passes=False, use_tc_tiling_on_sc=False),
    )
    return f(ytab, srcs, dsts, eids, *ea_list, ea6p)


def _reduce_body(p_ref, r_ref, o_ref):
    o_ref[...] = jnp.sum(p_ref[...], axis=0) + r_ref[...]


def _reduce_kernel(partials, rootb):
    bc = 7168
    return pl.pallas_call(
        _reduce_body,
        grid=(AGG // bc,),
        in_specs=[pl.BlockSpec((NW, bc), lambda j: (0, j)),
                  pl.BlockSpec((bc,), lambda j: (j,))],
        out_specs=pl.BlockSpec((bc,), lambda j: (j,)),
        out_shape=jax.ShapeDtypeStruct((AGG,), jnp.float32),
    )(partials, rootb)


def kernel(x_ind, x_org, x_ext, edge_indexes, edge_attrs, params):
    w_src = [jnp.zeros((D, WCOLS), jnp.float32) for _ in range(3)]
    for k in range(NREL):
        w_nn = params[k][0]
        sid, slot = REL_SRC_ID[k], REL_SLOT[k]
        de = w_nn.shape[0]
        w_src[sid] = w_src[sid].at[:, slot * DE: slot * DE + de].set(w_nn.T)
    roots = jnp.concatenate([params[k][2] for k in range(NREL)], axis=1)
    w_src[0] = w_src[0].at[:, 2 * DE: 2 * DE + NREL].set(roots)

    y_ind = _mm(x_ind, w_src[0])
    y_org = _mm(x_org, w_src[1])
    y_ext = _mm(x_ext, w_src[2])
    ytab = jnp.concatenate([y_ind, y_org, y_ext], axis=0).reshape(9 * N_NODE, DE)

    bias = jnp.concatenate([params[k][3] for k in range(NREL)])
    rootb = (y_ind[:, 2 * DE: 2 * DE + NREL] + bias[None, :]).reshape(-1)
    rootb = jnp.pad(rootb, (0, AGG - NREL * N_NODE))

    srcs, dsts = [], []
    for k in range(NREL):
        ei = edge_indexes[k]
        src_adj = REL_SRC_ID[k] * 3 * N_NODE + ei[0] * 3 + REL_SLOT[k]
        dst_adj = ei[1] * NREL + k
        srcs.append(jnp.pad(src_adj, (0, EPAD - E)))
        dsts.append(jnp.pad(dst_adj, (0, EPAD - E), constant_values=DEAD))
    srcs = jnp.concatenate(srcs)
    dsts = jnp.concatenate(dsts)
    eids = jnp.minimum(jnp.arange(EPAD, dtype=jnp.int32), E - 1)
    ea6p = jnp.pad(edge_attrs[6][:, 0], (0, EPAD - E))

    partials = _edge_kernel(ytab, srcs, dsts, eids, edge_attrs[:6], ea6p)

    out_flat = _reduce_kernel(partials, rootb)
    return out_flat[: NREL * N_NODE].reshape(N_NODE, NREL)


# speedup vs baseline: 6.0025x; 1.2563x over previous
import functools

import jax
import jax.numpy as jnp
from jax import lax
from jax.experimental import pallas as pl
from jax.experimental.pallas import tpu as pltpu
from jax.experimental.pallas import tpu_sc as plsc

N_NODE = 10000
E = 160000
D = 256
DE = 16
NREL = 7
NC, NS, L = 2, 16, 16
NW = NC * NS
PER_W = 5120
CE = 640
NCH = PER_W // CE
GPC = CE // L
AGG = 71680
DEAD = 70000

REL_SRC_ID = (0, 1, 2, 0, 1, 2, 1)
REL_SLOT = (0, 0, 0, 1, 1, 1, 2)
WCOLS = 48


def _mm_body(x_ref, w_ref, o_ref):
    o_ref[...] = lax.dot_general(
        x_ref[...], w_ref[...], (((1,), (0,)), ((), ())),
        preferred_element_type=jnp.float32,
        precision=lax.Precision.HIGHEST)


def _mm(x, w):
    br = 2000
    return pl.pallas_call(
        _mm_body,
        grid=(N_NODE // br,),
        in_specs=[pl.BlockSpec((br, D), lambda r: (r, 0)),
                  pl.BlockSpec((D, WCOLS), lambda r: (0, 0))],
        out_specs=pl.BlockSpec((br, WCOLS), lambda r: (r, 0)),
        out_shape=jax.ShapeDtypeStruct((N_NODE, WCOLS), jnp.float32),
    )(x, w)


def _edge_body(yi, yo, ye, sidx, ei0, ei1, ei2, ei3, ei4, ei5, ei6,
               ea0, ea1, ea2, ea3, ea4, ea5, ea6f, out,
               si_a, si_b, dr_a, dr_b, ea_a, ea_b, rw_a, rw_b, ea6_a,
               ea6_b, agg_v, sp_a, sp_b, sg_a, sg_b):
    ytabs = (yi, yo, ye)
    eis = (ei0, ei1, ei2, ei3, ei4, ei5, ei6)
    eas = (ea0, ea1, ea2, ea3, ea4, ea5, ea6f)
    c_ax = lax.axis_index("c")
    s_ax = lax.axis_index("s")
    wid = s_ax * NC + c_ax
    iota = lax.iota(jnp.int32, L)

    def zero_body(g, carry):
        agg_v[pl.ds(g * L, L)] = jnp.zeros((L,), jnp.float32)
        return carry
    lax.fori_loop(0, AGG // L, zero_body, 0)

    def win_base(ci):
        return jnp.minimum(wid * PER_W + ci * CE, E - CE)

    def issue_lin(k, ci, sibuf, drbuf, eabuf, ea6buf, sem):
        b = win_base(ci)
        pltpu.async_copy(sidx.at[pl.ds(k * E + b, CE)], sibuf, sem)
        pltpu.async_copy(eis[k].at[pl.ds(E + b, CE)], drbuf, sem)
        if k < 6:
            pltpu.async_copy(eas[k].at[pl.ds(b, CE), :], eabuf, sem)
        else:
            pltpu.async_copy(eas[6].at[pl.ds(b, CE)], ea6buf, sem)

    def wait_lin(k, sibuf, drbuf, eabuf, ea6buf, sem):
        pltpu.make_async_copy(sidx.at[pl.ds(0, CE)], sibuf, sem).wait()
        pltpu.make_async_copy(eis[k].at[pl.ds(0, CE)], drbuf, sem).wait()
        if k < 6:
            pltpu.make_async_copy(eas[k].at[pl.ds(0, CE), :],
                                  eabuf, sem).wait()
        else:
            pltpu.make_async_copy(eas[6].at[pl.ds(0, CE)],
                                  ea6buf, sem).wait()

    def issue_yg(k, sibuf, rwbuf, sem):
        pltpu.async_copy(ytabs[REL_SRC_ID[k]].at[sibuf], rwbuf, sem)

    def wait_yg(k, sibuf, rwbuf, sem):
        pltpu.make_async_copy(ytabs[REL_SRC_ID[k]].at[sibuf],
                              rwbuf, sem).wait()

    def compute(k, ci, drbuf, eabuf, ea6buf, rwbuf):
        b = win_base(ci)
        thr = wid * PER_W + ci * CE - b
        dead = jnp.full((L,), DEAD, jnp.int32)

        def group_body(g, carry):
            row0 = g * L
            ridx = row0 + iota
            if k < 6:
                acc = jnp.zeros((L,), jnp.float32)
                for j in range(DE):
                    cidx = jnp.full((L,), j, jnp.int32)
                    yv = plsc.load_gather(rwbuf, [ridx, cidx])
                    ev = plsc.load_gather(eabuf, [ridx, cidx])
                    acc = acc + yv * ev
            else:
                cidx = jnp.zeros((L,), jnp.int32)
                yv = plsc.load_gather(rwbuf, [ridx, cidx])
                acc = yv * ea6buf[pl.ds(row0, L)]
            dadj = drbuf[pl.ds(row0, L)] * NREL + k
            dv = jnp.where(ridx >= thr, dadj, dead)
            plsc.addupdate_scatter(agg_v, [dv], acc)
            return carry
        lax.fori_loop(0, GPC, group_body, 0)

    for k in range(NREL):
        issue_lin(k, 0, si_a, dr_a, ea_a, ea6_a, sp_a)
        issue_lin(k, 1, si_b, dr_b, ea_b, ea6_b, sp_b)
        wait_lin(k, si_a, dr_a, ea_a, ea6_a, sp_a)
        issue_yg(k, si_a, rw_a, sg_a)

        def pair_body(i, carry, k=k):
            c0 = 2 * i
            wait_lin(k, si_b, dr_b, ea_b, ea6_b, sp_b)
            issue_yg(k, si_b, rw_b, sg_b)
            wait_yg(k, si_a, rw_a, sg_a)
            compute(k, c0, dr_a, ea_a, ea6_a, rw_a)
            issue_lin(k, c0 + 2, si_a, dr_a, ea_a, ea6_a, sp_a)
            wait_lin(k, si_a, dr_a, ea_a, ea6_a, sp_a)
            issue_yg(k, si_a, rw_a, sg_a)
            wait_yg(k, si_b, rw_b, sg_b)
            compute(k, c0 + 1, dr_b, ea_b, ea6_b, rw_b)
            issue_lin(k, c0 + 3, si_b, dr_b, ea_b, ea6_b, sp_b)
            return carry
        lax.fori_loop(0, (NCH - 2) // 2, pair_body, 0)

        wait_lin(k, si_b, dr_b, ea_b, ea6_b, sp_b)
        issue_yg(k, si_b, rw_b, sg_b)
        wait_yg(k, si_a, rw_a, sg_a)
        compute(k, NCH - 2, dr_a, ea_a, ea6_a, rw_a)
        wait_yg(k, si_b, rw_b, sg_b)
        compute(k, NCH - 1, dr_b, ea_b, ea6_b, rw_b)

    pltpu.sync_copy(agg_v, out.at[wid])


def _edge_kernel(ytabs, sidx, eis, eas):
    mesh = plsc.VectorSubcoreMesh(
        core_axis_name="c", subcore_axis_name="s",
        num_cores=NC, num_subcores=NS)
    f = pl.kernel(
        _edge_body,
        out_type=jax.ShapeDtypeStruct((NW, AGG), jnp.float32),
        mesh=mesh,
        scratch_types=[
            pltpu.VMEM((CE,), jnp.int32),
            pltpu.VMEM((CE,), jnp.int32),
            pltpu.VMEM((CE,), jnp.int32),
            pltpu.VMEM((CE,), jnp.int32),
            pltpu.VMEM((CE, DE), jnp.float32),
            pltpu.VMEM((CE, DE), jnp.float32),
            pltpu.VMEM((CE, DE), jnp.float32),
            pltpu.VMEM((CE, DE), jnp.float32),
            pltpu.VMEM((CE,), jnp.float32),
            pltpu.VMEM((CE,), jnp.float32),
            pltpu.VMEM((AGG,), jnp.float32),
            pltpu.SemaphoreType.DMA,
            pltpu.SemaphoreType.DMA,
            pltpu.SemaphoreType.DMA,
            pltpu.SemaphoreType.DMA,
        ],
        compiler_params=pltpu.CompilerParams(
            needs_layout_passes=False, use_tc_tiling_on_sc=False),
    )
    return f(*ytabs, sidx, *eis, *eas)


def _reduce_body(p_ref, r_ref, o_ref):
    o_ref[...] = jnp.sum(p_ref[...], axis=0) + r_ref[...]


def _reduce_kernel(partials, rootb):
    bc = 7168
    return pl.pallas_call(
        _reduce_body,
        grid=(AGG // bc,),
        in_specs=[pl.BlockSpec((NW, bc), lambda j: (0, j)),
                  pl.BlockSpec((bc,), lambda j: (j,))],
        out_specs=pl.BlockSpec((bc,), lambda j: (j,)),
        out_shape=jax.ShapeDtypeStruct((AGG,), jnp.float32),
    )(partials, rootb)


def kernel(x_ind, x_org, x_ext, edge_indexes, edge_attrs, params):
    w_src = [jnp.zeros((D, WCOLS), jnp.float32) for _ in range(3)]
    for k in range(NREL):
        w_nn = params[k][0]
        sid, slot = REL_SRC_ID[k], REL_SLOT[k]
        de = w_nn.shape[0]
        w_src[sid] = w_src[sid].at[:, slot * DE: slot * DE + de].set(w_nn.T)
    roots = jnp.concatenate([params[k][2] for k in range(NREL)], axis=1)
    w_src[0] = w_src[0].at[:, 2 * DE: 2 * DE + NREL].set(roots)

    y2d = [_mm(x_ind, w_src[0]), _mm(x_org, w_src[1]), _mm(x_ext, w_src[2])]
    ytabs = [y.reshape(3 * N_NODE, DE) for y in y2d]

    bias = jnp.concatenate([params[k][3] for k in range(NREL)])
    rootb = (y2d[0][:, 2 * DE: 2 * DE + NREL] + bias[None, :]).reshape(-1)
    rootb = jnp.pad(rootb, (0, AGG - NREL * N_NODE))

    sidx = jnp.concatenate(
        [edge_indexes[k][0] * 3 + REL_SLOT[k] for k in range(NREL)])
    eis = [ei.reshape(2 * E) for ei in edge_indexes]
    eas = [edge_attrs[k] for k in range(6)] + [edge_attrs[6].reshape(E)]

    partials = _edge_kernel(ytabs, sidx, eis, eas)

    out_flat = _reduce_kernel(partials, rootb)
    return out_flat[: NREL * N_NODE].reshape(N_NODE, NREL)


# speedup vs baseline: 6.5137x; 1.0852x over previous
import functools

import jax
import jax.numpy as jnp
from jax import lax
from jax.experimental import pallas as pl
from jax.experimental.pallas import tpu as pltpu
from jax.experimental.pallas import tpu_sc as plsc

N_NODE = 10000
E = 160000
D = 256
DE = 16
NREL = 7
NC, NS, L = 2, 16, 16
NW = NC * NS
PER_W = 5120
CE = 640
NCH = PER_W // CE
GPC = CE // L
AGG = 71680
DEAD = 70000

REL_SRC_ID = (0, 1, 2, 0, 1, 2, 1)
REL_SLOT = (0, 0, 0, 1, 1, 1, 2)
WCOLS = 48


def _mm_body(x_ref, w_ref, o_ref):
    o_ref[...] = lax.dot_general(
        x_ref[...], w_ref[...], (((1,), (0,)), ((), ())),
        preferred_element_type=jnp.float32,
        precision=lax.Precision.HIGHEST)


def _mm(x, w):
    br = 2000
    return pl.pallas_call(
        _mm_body,
        grid=(N_NODE // br,),
        in_specs=[pl.BlockSpec((br, D), lambda r: (r, 0)),
                  pl.BlockSpec((D, WCOLS), lambda r: (0, 0))],
        out_specs=pl.BlockSpec((br, WCOLS), lambda r: (r, 0)),
        out_shape=jax.ShapeDtypeStruct((N_NODE, WCOLS), jnp.float32),
    )(x, w)


def _edge_body(yi, yo, ye,
               sa0, sa1, sa2, sa3, sa4, sa5, sa6,
               da0, da1, da2, da3, da4, da5, da6,
               ea0, ea1, ea2, ea3, ea4, ea5, ea6f, out,
               si_a, si_b, dr_a, dr_b, ea_a, ea_b, rw_a, rw_b, ea6_a,
               ea6_b, agg_v, sp_a, sp_b, sg_a, sg_b):
    ytabs = (yi, yo, ye)
    sas = (sa0, sa1, sa2, sa3, sa4, sa5, sa6)
    das = (da0, da1, da2, da3, da4, da5, da6)
    eas = (ea0, ea1, ea2, ea3, ea4, ea5, ea6f)
    c_ax = lax.axis_index("c")
    s_ax = lax.axis_index("s")
    wid = s_ax * NC + c_ax
    iota = lax.iota(jnp.int32, L)

    def zero_body(g, carry):
        agg_v[pl.ds(g * L, L)] = jnp.zeros((L,), jnp.float32)
        return carry
    lax.fori_loop(0, AGG // L, zero_body, 0)

    def win_base(ci):
        return jnp.minimum(wid * PER_W + ci * CE, E - CE)

    def issue_lin(k, ci, sibuf, drbuf, eabuf, ea6buf, sem):
        b = win_base(ci)
        pltpu.async_copy(sas[k].at[pl.ds(b, CE)], sibuf, sem)
        pltpu.async_copy(das[k].at[pl.ds(b, CE)], drbuf, sem)
        if k < 6:
            pltpu.async_copy(eas[k].at[pl.ds(b * DE, CE * DE)], eabuf, sem)
        else:
            pltpu.async_copy(eas[6].at[pl.ds(b, CE)], ea6buf, sem)

    def wait_lin(k, sibuf, drbuf, eabuf, ea6buf, sem):
        pltpu.make_async_copy(sas[k].at[pl.ds(0, CE)], sibuf, sem).wait()
        pltpu.make_async_copy(das[k].at[pl.ds(0, CE)], drbuf, sem).wait()
        if k < 6:
            pltpu.make_async_copy(eas[k].at[pl.ds(0, CE * DE)],
                                  eabuf, sem).wait()
        else:
            pltpu.make_async_copy(eas[6].at[pl.ds(0, CE)],
                                  ea6buf, sem).wait()

    def issue_yg(k, sibuf, rwbuf, sem):
        pltpu.async_copy(ytabs[REL_SRC_ID[k]].at[sibuf], rwbuf, sem)

    def wait_yg(k, sibuf, rwbuf, sem):
        pltpu.make_async_copy(ytabs[REL_SRC_ID[k]].at[sibuf],
                              rwbuf, sem).wait()

    def compute(k, ci, drbuf, eabuf, ea6buf, rwbuf):
        b = win_base(ci)
        thr = wid * PER_W + ci * CE - b
        dead = jnp.full((L,), DEAD, jnp.int32)

        def group_body(g, carry):
            row0 = g * L
            ridx = row0 + iota
            if k < 6:
                acc = jnp.zeros((L,), jnp.float32)
                fidx = ridx * DE
                for j in range(DE):
                    cidx = jnp.full((L,), j, jnp.int32)
                    yv = plsc.load_gather(rwbuf, [ridx, cidx])
                    ev = plsc.load_gather(eabuf, [fidx + j])
                    acc = acc + yv * ev
            else:
                cidx = jnp.zeros((L,), jnp.int32)
                yv = plsc.load_gather(rwbuf, [ridx, cidx])
                acc = yv * ea6buf[pl.ds(row0, L)]
            dadj = drbuf[pl.ds(row0, L)] * NREL + k
            dv = jnp.where(ridx >= thr, dadj, dead)
            plsc.addupdate_scatter(agg_v, [dv], acc)
            return carry
        lax.fori_loop(0, GPC, group_body, 0)

    for k in range(NREL):
        issue_lin(k, 0, si_a, dr_a, ea_a, ea6_a, sp_a)
        issue_lin(k, 1, si_b, dr_b, ea_b, ea6_b, sp_b)
        wait_lin(k, si_a, dr_a, ea_a, ea6_a, sp_a)
        issue_yg(k, si_a, rw_a, sg_a)

        def pair_body(i, carry, k=k):
            c0 = 2 * i
            wait_lin(k, si_b, dr_b, ea_b, ea6_b, sp_b)
            issue_yg(k, si_b, rw_b, sg_b)
            wait_yg(k, si_a, rw_a, sg_a)
            compute(k, c0, dr_a, ea_a, ea6_a, rw_a)
            issue_lin(k, c0 + 2, si_a, dr_a, ea_a, ea6_a, sp_a)
            wait_lin(k, si_a, dr_a, ea_a, ea6_a, sp_a)
            issue_yg(k, si_a, rw_a, sg_a)
            wait_yg(k, si_b, rw_b, sg_b)
            compute(k, c0 + 1, dr_b, ea_b, ea6_b, rw_b)
            issue_lin(k, c0 + 3, si_b, dr_b, ea_b, ea6_b, sp_b)
            return carry
        lax.fori_loop(0, (NCH - 2) // 2, pair_body, 0)

        wait_lin(k, si_b, dr_b, ea_b, ea6_b, sp_b)
        issue_yg(k, si_b, rw_b, sg_b)
        wait_yg(k, si_a, rw_a, sg_a)
        compute(k, NCH - 2, dr_a, ea_a, ea6_a, rw_a)
        wait_yg(k, si_b, rw_b, sg_b)
        compute(k, NCH - 1, dr_b, ea_b, ea6_b, rw_b)

    pltpu.sync_copy(agg_v, out.at[wid])


def _edge_kernel(ytabs, sadjs, drs, eas):
    mesh = plsc.VectorSubcoreMesh(
        core_axis_name="c", subcore_axis_name="s",
        num_cores=NC, num_subcores=NS)
    f = pl.kernel(
        _edge_body,
        out_type=jax.ShapeDtypeStruct((NW, AGG), jnp.float32),
        mesh=mesh,
        scratch_types=[
            pltpu.VMEM((CE,), jnp.int32),
            pltpu.VMEM((CE,), jnp.int32),
            pltpu.VMEM((CE,), jnp.int32),
            pltpu.VMEM((CE,), jnp.int32),
            pltpu.VMEM((CE * DE,), jnp.float32),
            pltpu.VMEM((CE * DE,), jnp.float32),
            pltpu.VMEM((CE, DE), jnp.float32),
            pltpu.VMEM((CE, DE), jnp.float32),
            pltpu.VMEM((CE,), jnp.float32),
            pltpu.VMEM((CE,), jnp.float32),
            pltpu.VMEM((AGG,), jnp.float32),
            pltpu.SemaphoreType.DMA,
            pltpu.SemaphoreType.DMA,
            pltpu.SemaphoreType.DMA,
            pltpu.SemaphoreType.DMA,
        ],
        compiler_params=pltpu.CompilerParams(
            needs_layout_passes=False, use_tc_tiling_on_sc=False),
    )
    return f(*ytabs, *sadjs, *drs, *eas)


def _prep_body(slot, ei_ref, s_ref, d_ref):
    s_ref[...] = ei_ref[0, :] * 3 + slot
    d_ref[...] = ei_ref[1, :]


def _prep(ei, slot):
    return pl.pallas_call(
        functools.partial(_prep_body, slot),
        in_specs=[pl.BlockSpec((2, E), lambda: (0, 0))],
        out_specs=(pl.BlockSpec((E,), lambda: (0,)),
                   pl.BlockSpec((E,), lambda: (0,))),
        out_shape=(jax.ShapeDtypeStruct((E,), jnp.int32),
                   jax.ShapeDtypeStruct((E,), jnp.int32)),
    )(ei)


def _reduce_body(p_ref, r_ref, o_ref):
    o_ref[...] = jnp.sum(p_ref[...], axis=0) + r_ref[...]


def _reduce_kernel(partials, rootb):
    bc = 7168
    return pl.pallas_call(
        _reduce_body,
        grid=(AGG // bc,),
        in_specs=[pl.BlockSpec((NW, bc), lambda j: (0, j)),
                  pl.BlockSpec((bc,), lambda j: (j,))],
        out_specs=pl.BlockSpec((bc,), lambda j: (j,)),
        out_shape=jax.ShapeDtypeStruct((AGG,), jnp.float32),
    )(partials, rootb)


def kernel(x_ind, x_org, x_ext, edge_indexes, edge_attrs, params):
    w_src = [jnp.zeros((D, WCOLS), jnp.float32) for _ in range(3)]
    for k in range(NREL):
        w_nn = params[k][0]
        sid, slot = REL_SRC_ID[k], REL_SLOT[k]
        de = w_nn.shape[0]
        w_src[sid] = w_src[sid].at[:, slot * DE: slot * DE + de].set(w_nn.T)
    roots = jnp.concatenate([params[k][2] for k in range(NREL)], axis=1)
    w_src[0] = w_src[0].at[:, 2 * DE: 2 * DE + NREL].set(roots)

    y2d = [_mm(x_ind, w_src[0]), _mm(x_org, w_src[1]), _mm(x_ext, w_src[2])]
    ytabs = [y.reshape(3 * N_NODE, DE) for y in y2d]

    bias = jnp.concatenate([params[k][3] for k in range(NREL)])
    rootb = (y2d[0][:, 2 * DE: 2 * DE + NREL] + bias[None, :]).reshape(-1)
    rootb = jnp.pad(rootb, (0, AGG - NREL * N_NODE))

    preps = [_prep(edge_indexes[k], REL_SLOT[k]) for k in range(NREL)]
    sadjs = [p[0] for p in preps]
    drs = [p[1] for p in preps]
    eas = ([edge_attrs[k].reshape(E * DE) for k in range(6)]
           + [edge_attrs[6].reshape(E)])

    partials = _edge_kernel(ytabs, sadjs, drs, eas)

    out_flat = _reduce_kernel(partials, rootb)
    return out_flat[: NREL * N_NODE].reshape(N_NODE, NREL)


# speedup vs baseline: 11.2114x; 1.7212x over previous
import functools

import jax
import jax.numpy as jnp
from jax import lax
from jax.experimental import pallas as pl
from jax.experimental.pallas import tpu as pltpu
from jax.experimental.pallas import tpu_sc as plsc

N_NODE = 10000
E = 160000
D = 256
DE = 16
NREL = 7
NC, NS, L = 2, 16, 16
NW = NC * NS
PER_W = 5120
CE = 640
NCH = PER_W // CE
GPC = CE // L
AGG = 71680
DEAD = 70000

REL_SRC_ID = (0, 1, 2, 0, 1, 2, 1)
REL_SLOT = (0, 0, 0, 1, 1, 1, 2)
WCOLS = 48


def _mm_body(x_ref, w_ref, o_ref):
    o_ref[...] = lax.dot_general(
        x_ref[...], w_ref[...], (((1,), (0,)), ((), ())),
        preferred_element_type=jnp.float32,
        precision=lax.Precision.HIGHEST)


def _mm(x, w):
    br = 2000
    return pl.pallas_call(
        _mm_body,
        grid=(N_NODE // br,),
        in_specs=[pl.BlockSpec((br, D), lambda r: (r, 0)),
                  pl.BlockSpec((D, WCOLS), lambda r: (0, 0))],
        out_specs=pl.BlockSpec((br, WCOLS), lambda r: (r, 0)),
        out_shape=jax.ShapeDtypeStruct((N_NODE, WCOLS), jnp.float32),
    )(x, w)


def _edge_body(rels, *args):
    nr = len(rels)
    ytabs = args[0:3]
    sas = args[3:3 + nr]
    das = args[3 + nr:3 + 2 * nr]
    eas = args[3 + 2 * nr:3 + 3 * nr]
    out = args[3 + 3 * nr]
    (si_a, si_b, dr_a, dr_b, ea_a, ea_b, rw_a, rw_b, ea6_a,
     ea6_b, agg_v, sp_a, sp_b, sg_a, sg_b) = args[4 + 3 * nr:]
    c_ax = lax.axis_index("c")
    s_ax = lax.axis_index("s")
    wid = s_ax * NC + c_ax
    iota = lax.iota(jnp.int32, L)

    def zero_body(g, carry):
        agg_v[pl.ds(g * L, L)] = jnp.zeros((L,), jnp.float32)
        return carry
    lax.fori_loop(0, AGG // L, zero_body, 0)

    def win_base(ci):
        return jnp.minimum(wid * PER_W + ci * CE, E - CE)

    def issue_lin(li, k, ci, sibuf, drbuf, eabuf, ea6buf, sem):
        b = win_base(ci)
        pltpu.async_copy(sas[li].at[pl.ds(b, CE)], sibuf, sem)
        pltpu.async_copy(das[li].at[pl.ds(b, CE)], drbuf, sem)
        if k < 6:
            pltpu.async_copy(eas[li].at[:, pl.ds(b, CE)], eabuf, sem)
        else:
            pltpu.async_copy(eas[li].at[pl.ds(b, CE)], ea6buf, sem)

    def wait_lin(li, k, sibuf, drbuf, eabuf, ea6buf, sem):
        pltpu.make_async_copy(sas[li].at[pl.ds(0, CE)], sibuf, sem).wait()
        pltpu.make_async_copy(das[li].at[pl.ds(0, CE)], drbuf, sem).wait()
        if k < 6:
            pltpu.make_async_copy(eas[li].at[:, pl.ds(0, CE)],
                                  eabuf, sem).wait()
        else:
            pltpu.make_async_copy(eas[li].at[pl.ds(0, CE)],
                                  ea6buf, sem).wait()

    def issue_yg(k, sibuf, rwbuf, sem):
        pltpu.async_copy(ytabs[REL_SRC_ID[k]].at[sibuf], rwbuf, sem)

    def wait_yg(k, sibuf, rwbuf, sem):
        pltpu.make_async_copy(ytabs[REL_SRC_ID[k]].at[sibuf],
                              rwbuf, sem).wait()

    def compute(k, ci, drbuf, eabuf, ea6buf, rwbuf):
        b = win_base(ci)
        thr = wid * PER_W + ci * CE - b
        dead = jnp.full((L,), DEAD, jnp.int32)

        def group_body(g, carry):
            row0 = g * L
            ridx = row0 + iota
            if k < 6:
                acc = jnp.zeros((L,), jnp.float32)
                for j in range(DE):
                    cidx = jnp.full((L,), j, jnp.int32)
                    yv = plsc.load_gather(rwbuf, [ridx, cidx])
                    ev = eabuf[j, pl.ds(row0, L)]
                    acc = acc + yv * ev
            else:
                cidx = jnp.zeros((L,), jnp.int32)
                yv = plsc.load_gather(rwbuf, [ridx, cidx])
                acc = yv * ea6buf[pl.ds(row0, L)]
            dadj = drbuf[pl.ds(row0, L)] * NREL + k
            dv = jnp.where(ridx >= thr, dadj, dead)
            plsc.addupdate_scatter(agg_v, [dv], acc)
            return carry
        lax.fori_loop(0, GPC, group_body, 0)

    for li, k in enumerate(rels):
        issue_lin(li, k, 0, si_a, dr_a, ea_a, ea6_a, sp_a)
        issue_lin(li, k, 1, si_b, dr_b, ea_b, ea6_b, sp_b)
        wait_lin(li, k, si_a, dr_a, ea_a, ea6_a, sp_a)
        issue_yg(k, si_a, rw_a, sg_a)

        def pair_body(i, carry, li=li, k=k):
            c0 = 2 * i
            wait_lin(li, k, si_b, dr_b, ea_b, ea6_b, sp_b)
            issue_yg(k, si_b, rw_b, sg_b)
            wait_yg(k, si_a, rw_a, sg_a)
            compute(k, c0, dr_a, ea_a, ea6_a, rw_a)
            issue_lin(li, k, c0 + 2, si_a, dr_a, ea_a, ea6_a, sp_a)
            wait_lin(li, k, si_a, dr_a, ea_a, ea6_a, sp_a)
            issue_yg(k, si_a, rw_a, sg_a)
            wait_yg(k, si_b, rw_b, sg_b)
            compute(k, c0 + 1, dr_b, ea_b, ea6_b, rw_b)
            issue_lin(li, k, c0 + 3, si_b, dr_b, ea_b, ea6_b, sp_b)
            return carry
        lax.fori_loop(0, (NCH - 2) // 2, pair_body, 0)

        wait_lin(li, k, si_b, dr_b, ea_b, ea6_b, sp_b)
        issue_yg(k, si_b, rw_b, sg_b)
        wait_yg(k, si_a, rw_a, sg_a)
        compute(k, NCH - 2, dr_a, ea_a, ea6_a, rw_a)
        wait_yg(k, si_b, rw_b, sg_b)
        compute(k, NCH - 1, dr_b, ea_b, ea6_b, rw_b)

    pltpu.sync_copy(agg_v, out.at[wid])


def _edge_kernel(rels, ytabs, sadjs, drs, eas):
    mesh = plsc.VectorSubcoreMesh(
        core_axis_name="c", subcore_axis_name="s",
        num_cores=NC, num_subcores=NS)
    f = pl.kernel(
        functools.partial(_edge_body, rels),
        out_type=jax.ShapeDtypeStruct((NW, AGG), jnp.float32),
        mesh=mesh,
        scratch_types=[
            pltpu.VMEM((CE,), jnp.int32),
            pltpu.VMEM((CE,), jnp.int32),
            pltpu.VMEM((CE,), jnp.int32),
            pltpu.VMEM((CE,), jnp.int32),
            pltpu.VMEM((DE, CE), jnp.float32),
            pltpu.VMEM((DE, CE), jnp.float32),
            pltpu.VMEM((CE, DE), jnp.float32),
            pltpu.VMEM((CE, DE), jnp.float32),
            pltpu.VMEM((CE,), jnp.float32),
            pltpu.VMEM((CE,), jnp.float32),
            pltpu.VMEM((AGG,), jnp.float32),
            pltpu.SemaphoreType.DMA,
            pltpu.SemaphoreType.DMA,
            pltpu.SemaphoreType.DMA,
            pltpu.SemaphoreType.DMA,
        ],
        compiler_params=pltpu.CompilerParams(
            needs_layout_passes=False, use_tc_tiling_on_sc=False),
    )
    return f(*ytabs,
             *[sadjs[k] for k in rels],
             *[drs[k] for k in rels],
             *[eas[k] for k in rels])


def _prep_body(slot, ei_ref, s_ref, d_ref):
    s_ref[...] = ei_ref[0, :] * 3 + slot
    d_ref[...] = ei_ref[1, :]


def _prep(ei, slot):
    return pl.pallas_call(
        functools.partial(_prep_body, slot),
        in_specs=[pl.BlockSpec((2, E), lambda: (0, 0))],
        out_specs=(pl.BlockSpec((E,), lambda: (0,)),
                   pl.BlockSpec((E,), lambda: (0,))),
        out_shape=(jax.ShapeDtypeStruct((E,), jnp.int32),
                   jax.ShapeDtypeStruct((E,), jnp.int32)),
    )(ei)


def _reduce_body(*refs):
    o_ref = refs[-1]
    r_ref = refs[-2]
    acc = r_ref[...]
    for p_ref in refs[:-2]:
        acc = acc + jnp.sum(p_ref[...], axis=0)
    o_ref[...] = acc


def _reduce_kernel(partials_list, rootb):
    bc = 7168
    return pl.pallas_call(
        _reduce_body,
        grid=(AGG // bc,),
        in_specs=([pl.BlockSpec((NW, bc), lambda j: (0, j))
                   for _ in partials_list]
                  + [pl.BlockSpec((bc,), lambda j: (j,))]),
        out_specs=pl.BlockSpec((bc,), lambda j: (j,)),
        out_shape=jax.ShapeDtypeStruct((AGG,), jnp.float32),
    )(*partials_list, rootb)


def kernel(x_ind, x_org, x_ext, edge_indexes, edge_attrs, params):
    w_src = [jnp.zeros((D, WCOLS), jnp.float32) for _ in range(3)]
    for k in range(NREL):
        w_nn = params[k][0]
        sid, slot = REL_SRC_ID[k], REL_SLOT[k]
        de = w_nn.shape[0]
        w_src[sid] = w_src[sid].at[:, slot * DE: slot * DE + de].set(w_nn.T)
    roots = jnp.concatenate([params[k][2] for k in range(NREL)], axis=1)
    w_src[0] = w_src[0].at[:, 2 * DE: 2 * DE + NREL].set(roots)

    y2d = [_mm(x_ind, w_src[0]), _mm(x_org, w_src[1]), _mm(x_ext, w_src[2])]
    ytabs = [y.reshape(3 * N_NODE, DE) for y in y2d]

    bias = jnp.concatenate([params[k][3] for k in range(NREL)])
    rootb = (y2d[0][:, 2 * DE: 2 * DE + NREL] + bias[None, :]).reshape(-1)
    rootb = jnp.pad(rootb, (0, AGG - NREL * N_NODE))

    preps = [_prep(edge_indexes[k], REL_SLOT[k]) for k in range(NREL)]
    sadjs = [p[0] for p in preps]
    drs = [p[1] for p in preps]
    eas = ([edge_attrs[k].T for k in range(6)]
           + [edge_attrs[6].reshape(E)])

    groups = [(0, 1), (2, 3), (4, 5), (6,)]
    partials_list = [_edge_kernel(g, ytabs, sadjs, drs, eas) for g in groups]

    out_flat = _reduce_kernel(partials_list, rootb)
    return out_flat[: NREL * N_NODE].reshape(N_NODE, NREL)
